# baseline (speedup 1.0000x reference)
import functools

import jax
import jax.numpy as jnp
from jax import lax
from jax.experimental import pallas as pl
from jax.experimental.pallas import tpu as pltpu
from jax.experimental.pallas import tpu_sc as plsc

NC = 2
NS = 16
NW = NC * NS
LANES = 16
C = 32
GI = 128


@functools.partial(jax.jit, static_argnames=("B", "D", "NEG"))
def _sc_dots(in_idx, ctx_idx, neg_idx, in_emb, out_emb, *, B, D, NEG):
    BPW = B // NW
    NCHUNK = BPW // C
    CN = C * NEG
    NSUB = CN // GI
    KD = D // LANES

    mesh = plsc.VectorSubcoreMesh(core_axis_name="c", subcore_axis_name="s",
                                  num_cores=NC, num_subcores=NS)

    idx_scratch = [pltpu.VMEM((C,), jnp.int32) for _ in range(4)]
    nidx_scratch = [pltpu.VMEM((GI,), jnp.int32) for _ in range(2 * NSUB)]
    row_scratch = [pltpu.VMEM((C, D), jnp.float32) for _ in range(4)]
    nrow_scratch = [pltpu.VMEM((GI, D), jnp.float32) for _ in range(2 * NSUB)]

    @functools.partial(
        pl.kernel,
        out_type=(
            jax.ShapeDtypeStruct((B,), jnp.float32),
            jax.ShapeDtypeStruct((B * NEG,), jnp.float32),
        ),
        mesh=mesh,
        compiler_params=pltpu.CompilerParams(needs_layout_passes=False,
                                             use_tc_tiling_on_sc=False),
        scratch_types=idx_scratch + nidx_scratch + row_scratch + nrow_scratch
        + [
            pltpu.VMEM((C,), jnp.float32),
            pltpu.VMEM((CN,), jnp.float32),
            pltpu.SemaphoreType.DMA,
            pltpu.SemaphoreType.DMA,
        ],
    )
    def k(in_idx_h, ctx_idx_h, neg_idx_h, in_emb_h, out_emb_h,
          pos_h, neg_h, *refs):
        ii_v = (refs[0], refs[1])
        ci_v = (refs[2], refs[3])
        ni_v = (refs[4:4 + NSUB], refs[4 + NSUB:4 + 2 * NSUB])
        o = 4 + 2 * NSUB
        ir_v = (refs[o], refs[o + 1])
        cr_v = (refs[o + 2], refs[o + 3])
        nr_v = (refs[o + 4:o + 4 + NSUB], refs[o + 4 + NSUB:o + 4 + 2 * NSUB])
        po_v, no_v, sem0, sem1 = refs[o + 4 + 2 * NSUB:]
        sems = (sem0, sem1)

        wid = lax.axis_index("s") * NC + lax.axis_index("c")
        lane = lax.iota(jnp.int32, LANES)
        lane_masks = [lane == l for l in range(LANES)]

        def gathers(b):
            sem = sems[b]
            out = [
                (in_emb_h.at[ii_v[b]], ir_v[b], sem),
                (out_emb_h.at[ci_v[b]], cr_v[b], sem),
            ]
            for s in range(NSUB):
                out.append((out_emb_h.at[ni_v[b][s]], nr_v[b][s], sem))
            return out

        def stage(cid, b):
            pltpu.sync_copy(in_idx_h.at[pl.ds(cid * C, C)], ii_v[b])
            pltpu.sync_copy(ctx_idx_h.at[pl.ds(cid * C, C)], ci_v[b])
            for s in range(NSUB):
                pltpu.sync_copy(
                    neg_idx_h.at[pl.ds(cid * CN + s * GI, GI)], ni_v[b][s])
            for src, dst, sem in gathers(b):
                pltpu.make_async_copy(src, dst, sem).start()

        def wait(b):
            for src, dst, sem in gathers(b):
                pltpu.make_async_copy(src, dst, sem).wait()

        def dot(row_chunks, in_chunks):
            acc = in_chunks[0] * row_chunks[0]
            for kk in range(1, KD):
                acc = acc + in_chunks[kk] * row_chunks[kk]
            return jnp.sum(acc)

        def pack16(scalars):
            vec = jnp.zeros((LANES,), jnp.float32)
            for l in range(LANES):
                vec = jnp.where(lane_masks[l], scalars[l], vec)
            return vec

        def compute(cid, b):
            for h in range(C // LANES):
                scal = []
                for l in range(LANES):
                    r = h * LANES + l
                    ivs = [ir_v[b][r, pl.ds(kk * LANES, LANES)]
                           for kk in range(KD)]
                    cvs = [cr_v[b][r, pl.ds(kk * LANES, LANES)]
                           for kk in range(KD)]
                    scal.append(dot(cvs, ivs))
                po_v[pl.ds(h * LANES, LANES)] = pack16(scal)

            for s in range(NSUB):
                @pl.loop(0, GI // LANES)
                def _(v, s=s):
                    scal = []
                    for l in range(LANES):
                        t = v * LANES + l
                        flat = s * GI + t
                        r = flat // NEG
                        ivs = [ir_v[b][r, pl.ds(kk * LANES, LANES)]
                               for kk in range(KD)]
                        nvs = [nr_v[b][s][t, pl.ds(kk * LANES, LANES)]
                               for kk in range(KD)]
                        scal.append(dot(nvs, ivs))
                    no_v[pl.ds(s * GI + v * LANES, LANES)] = pack16(scal)

            pltpu.sync_copy(po_v, pos_h.at[pl.ds(cid * C, C)])
            pltpu.sync_copy(no_v, neg_h.at[pl.ds(cid * CN, CN)])

        first = wid * NCHUNK
        stage(first, 0)

        @pl.loop(0, NCHUNK, step=2)
        def _(g2):
            cid0 = first + g2
            stage(cid0 + 1, 1)
            wait(0)
            compute(cid0, 0)

            @pl.when(g2 + 2 < NCHUNK)
            def _():
                stage(cid0 + 2, 0)

            wait(1)
            compute(cid0 + 1, 1)

    return k(in_idx, ctx_idx, neg_idx, in_emb, out_emb)


def _loss_body(pos_ref, neg_ref, o_ref, *, B):
    def ls(x):
        return jnp.minimum(x, 0.0) - jnp.log1p(jnp.exp(-jnp.abs(x)))

    tot = jnp.sum(ls(pos_ref[...])) + jnp.sum(ls(-neg_ref[...]))
    o_ref[0, 0] = -tot / B


@functools.partial(jax.jit, static_argnames=("B",))
def _tc_loss(pos2d, neg2d, *, B):
    return pl.pallas_call(
        functools.partial(_loss_body, B=B),
        out_shape=jax.ShapeDtypeStruct((1, 1), jnp.float32),
        out_specs=pl.BlockSpec(memory_space=pltpu.SMEM),
    )(pos2d, neg2d)


def kernel(inputs, contexts, negatives, in_emb, out_emb):
    B, NEG = negatives.shape
    V, D = in_emb.shape
    in_idx = inputs.reshape(B)
    ctx_idx = contexts.reshape(B)
    neg_idx = negatives.reshape(B * NEG)
    pos, neg = _sc_dots(in_idx, ctx_idx, neg_idx, in_emb, out_emb,
                        B=B, D=D, NEG=NEG)
    loss = _tc_loss(pos.reshape(B // 128, 128), neg.reshape(-1, 128), B=B)
    return loss[0, 0]
